# baseline (speedup 1.0000x reference)
import functools

import jax
import jax.numpy as jnp
from jax import lax
from jax.experimental import pallas as pl
from jax.experimental.pallas import tpu as pltpu
from jax.experimental.pallas import tpu_sc as plsc

B = 16384
D = 64
CH = 128


@functools.cache
def _build_kernel(n_rows):
    info = plsc.get_sparse_core_info()
    nw = info.num_cores * info.num_subcores
    b_per_w = B // nw
    n_ch = b_per_w // CH
    mesh = plsc.VectorSubcoreMesh(core_axis_name="c", subcore_axis_name="s")

    @functools.partial(
        pl.kernel,
        mesh=mesh,
        out_type=(
            jax.ShapeDtypeStruct((B, D), jnp.float32),
            jax.ShapeDtypeStruct((B, D), jnp.float32),
        ),
        scratch_types=[
            pltpu.VMEM((b_per_w,), jnp.int32),
            pltpu.VMEM((b_per_w,), jnp.int32),
            pltpu.VMEM((CH, 2 * D), jnp.float32),
            pltpu.VMEM((CH, 2 * D), jnp.float32),
            pltpu.VMEM((CH, D), jnp.float32),
            pltpu.VMEM((CH, D), jnp.float32),
            pltpu.SemaphoreType.DMA,
            pltpu.SemaphoreType.DMA,
            pltpu.SemaphoreType.DMA,
            pltpu.SemaphoreType.DMA,
        ],
        compiler_params=pltpu.CompilerParams(needs_layout_passes=False),
    )
    def k(ids_hbm, ws_hbm, wa_hbm, out_s_hbm, out_a_hbm,
          idx_v, pidx_v, rows_s, rows_a, sel_s, sel_a,
          sem_s, sem_a, sem_os, sem_oa):
        wid = lax.axis_index("s") * info.num_cores + lax.axis_index("c")
        base = wid * b_per_w
        pltpu.sync_copy(ids_hbm.at[pl.ds(base, b_per_w)], idx_v)
        iota = lax.iota(jnp.int32, 16)
        for j in range(b_per_w // 16):
            pidx_v[pl.ds(j * 16, 16)] = idx_v[pl.ds(j * 16, 16)] >> 1

        for ch in range(n_ch):
            cp_s = pltpu.async_copy(
                ws_hbm.at[pidx_v.at[pl.ds(ch * CH, CH)]], rows_s, sem_s)
            cp_a = pltpu.async_copy(
                wa_hbm.at[pidx_v.at[pl.ds(ch * CH, CH)]], rows_a, sem_a)
            cp_s.wait()
            cp_a.wait()

            def body(i, _):
                idsplat = plsc.load_gather(
                    idx_v, [jnp.full((16,), ch * CH, jnp.int32) + i])
                half = (idsplat & 1) << 6
                rowv = jnp.full((16,), 0, jnp.int32) + i
                for q in range(D // 16):
                    colv = half + (q * 16 + iota)
                    sel_s[i, pl.ds(q * 16, 16)] = plsc.load_gather(
                        rows_s, [rowv, colv])
                    sel_a[i, pl.ds(q * 16, 16)] = plsc.load_gather(
                        rows_a, [rowv, colv])
                return 0

            lax.fori_loop(0, CH, body, 0)
            pltpu.async_copy(
                sel_s, out_s_hbm.at[pl.ds(base + ch * CH, CH)], sem_os).wait()
            pltpu.async_copy(
                sel_a, out_a_hbm.at[pl.ds(base + ch * CH, CH)], sem_oa).wait()

    return k


def kernel(instance_ids, W_shape, W_appearance):
    ids = instance_ids.astype(jnp.int32)
    n = W_shape.shape[0]
    ws = W_shape.reshape(n // 2, 2 * D)
    wa = W_appearance.reshape(n // 2, 2 * D)
    return _build_kernel(n)(ids, ws, wa)


# speedup vs baseline: 1.6042x; 1.6042x over previous
import functools

import jax
import jax.numpy as jnp
from jax import lax
from jax.experimental import pallas as pl
from jax.experimental.pallas import tpu as pltpu
from jax.experimental.pallas import tpu_sc as plsc

B = 16384
D = 64
CH = 256


@functools.cache
def _build_kernel():
    info = plsc.get_sparse_core_info()
    nw = info.num_cores * info.num_subcores
    b_per_w = B // nw
    n_ch = b_per_w // CH
    mesh = plsc.VectorSubcoreMesh(core_axis_name="c", subcore_axis_name="s")

    @functools.partial(
        pl.kernel,
        mesh=mesh,
        out_type=(
            jax.ShapeDtypeStruct((B, D), jnp.float32),
            jax.ShapeDtypeStruct((B, D), jnp.float32),
        ),
        scratch_types=[
            pltpu.VMEM((CH,), jnp.int32),
            pltpu.VMEM((CH, D), jnp.float32),
            pltpu.VMEM((CH, D), jnp.float32),
            pltpu.SemaphoreType.DMA,
            pltpu.SemaphoreType.DMA,
            pltpu.SemaphoreType.DMA,
        ],
    )
    def k(ids_hbm, ws_hbm, wa_hbm, out_s_hbm, out_a_hbm,
          idx_v, rows_s, rows_a, sem_g, sem_os, sem_oa):
        wid = lax.axis_index("s") * info.num_cores + lax.axis_index("c")
        base = wid * b_per_w

        for ch in range(n_ch):
            pltpu.sync_copy(ids_hbm.at[pl.ds(base + ch * CH, CH)], idx_v)

            def fire(g, _):
                v = idx_v[pl.ds(g * 16, 16)]
                for l in range(16):
                    r = v[l]
                    i = g * 16 + l
                    pltpu.async_copy(
                        ws_hbm.at[pl.ds(r, 1)], rows_s.at[pl.ds(i, 1)], sem_g)
                    pltpu.async_copy(
                        wa_hbm.at[pl.ds(r, 1)], rows_a.at[pl.ds(i, 1)], sem_g)
                return 0

            lax.fori_loop(0, CH // 16, fire, 0)
            pltpu.make_async_copy(ws_hbm.at[pl.ds(0, CH)], rows_s, sem_g).wait()
            pltpu.make_async_copy(wa_hbm.at[pl.ds(0, CH)], rows_a, sem_g).wait()
            pltpu.async_copy(
                rows_s, out_s_hbm.at[pl.ds(base + ch * CH, CH)], sem_os).wait()
            pltpu.async_copy(
                rows_a, out_a_hbm.at[pl.ds(base + ch * CH, CH)], sem_oa).wait()

    return k


def kernel(instance_ids, W_shape, W_appearance):
    ids = instance_ids.astype(jnp.int32)
    return _build_kernel()(ids, W_shape, W_appearance)
